# baseline (speedup 1.0000x reference)
import functools

import jax
import jax.numpy as jnp
from jax import lax
from jax.experimental import pallas as pl
from jax.experimental.pallas import tpu as pltpu
from jax.experimental.pallas import tpu_sc as plsc

D = 256
NE = 8192
NROW = 8192
RT = 1024
CT = 512
NI = NROW // RT
NJ = NE // CT
BETA = 0.25


SUB = 64
NS = RT // SUB


def _dist_body(x_ref, e2x_ref, idx_ref, loss_ref, x2_ref, min_ref, arg_ref):
    i = pl.program_id(0)
    j = pl.program_id(1)

    @pl.when(j == 0)
    def _():
        xt = x_ref[...]
        x2 = jnp.sum(xt * xt, axis=1, keepdims=True)
        x2_ref[...] = jnp.transpose(x2)
        min_ref[...] = jnp.full((1, RT), jnp.inf, jnp.float32)

    e2x = e2x_ref[...]
    e2 = 0.25 * jnp.sum(e2x * e2x, axis=0, keepdims=True)
    e2t = jnp.transpose(e2)
    jcol = jnp.float32(j * CT)

    sim2 = lax.dot_general(e2x, x_ref[...], (((0,), (1,)), ((), ())),
                           preferred_element_type=jnp.float32)
    d = (x2_ref[...] + e2t) - sim2
    m = jnp.min(d, axis=0, keepdims=True)
    iota = lax.broadcasted_iota(jnp.int32, (CT, RT), 0).astype(jnp.float32)
    a = jnp.min(jnp.where(d == m, iota, jnp.float32(1e9)),
                axis=0, keepdims=True) + jcol
    prev_m = min_ref[...]
    upd = m < prev_m
    min_ref[...] = jnp.where(upd, m, prev_m)
    arg_ref[...] = jnp.where(upd, a, arg_ref[...])

    @pl.when(j == NJ - 1)
    def _():
        idx_ref[...] = arg_ref[...].astype(jnp.int32).reshape(1, 1, RT)
        tile_sum = jnp.sum(min_ref[...])
        prev = jnp.where(i == 0, jnp.zeros((1, 1), jnp.float32), loss_ref[...])
        tot = prev + tile_sum
        scale = jnp.float32((1.0 + BETA) / (NROW * D))
        loss_ref[...] = jnp.where(i == NI - 1, tot * scale, tot)


_dist_call = pl.pallas_call(
    _dist_body,
    grid=(NI, NJ),
    in_specs=[
        pl.BlockSpec((RT, D), lambda i, j: (i, 0)),
        pl.BlockSpec((D, CT), lambda i, j: (0, j)),
    ],
    out_specs=[
        pl.BlockSpec((1, 1, RT), lambda i, j: (i, 0, 0)),
        pl.BlockSpec((1, 1), lambda i, j: (0, 0)),
    ],
    out_shape=[
        jax.ShapeDtypeStruct((NI, 1, RT), jnp.int32),
        jax.ShapeDtypeStruct((1, 1), jnp.float32),
    ],
    scratch_shapes=[
        pltpu.VMEM((1, RT), jnp.float32),
        pltpu.VMEM((1, RT), jnp.float32),
        pltpu.VMEM((1, RT), jnp.float32),
    ],
    compiler_params=pltpu.CompilerParams(
        dimension_semantics=("arbitrary", "arbitrary"),
    ),
)


NW = 32
BPW = NROW // NW
CHUNK = 128
NCH = BPW // CHUNK

@functools.cache
def _sc_gather_call():
    mesh = plsc.VectorSubcoreMesh(core_axis_name="c", subcore_axis_name="s")

    @functools.partial(
        pl.kernel,
        mesh=mesh,
        out_type=jax.ShapeDtypeStruct((NROW, D), jnp.float32),
        scratch_types=[
            pltpu.VMEM((NCH, CHUNK), jnp.int32),
            pltpu.VMEM((BPW, D), jnp.float32),
            pltpu.SemaphoreType.DMA,
        ],
    )
    def _sc_gather(table_hbm, idx_hbm, out_hbm, idx_v, rows_v, sem):
        wid = lax.axis_index("s") * 2 + lax.axis_index("c")
        base = wid * BPW
        pltpu.sync_copy(idx_hbm.at[wid], idx_v)
        cps = [
            pltpu.async_copy(table_hbm.at[idx_v.at[k]],
                             rows_v.at[pl.ds(k * CHUNK, CHUNK)], sem)
            for k in range(NCH)
        ]
        for cp in cps:
            cp.wait()
        pltpu.sync_copy(rows_v, out_hbm.at[pl.ds(base, BPW)])

    return _sc_gather


def kernel(x, embeddings):
    input_shape = x.shape
    xf = jnp.reshape(x, (NROW, D))
    idx3, loss2 = _dist_call(xf, embeddings + embeddings)
    idx = jnp.reshape(idx3, (NW, NCH, CHUNK))
    table = embeddings.T
    q = _sc_gather_call()(table, idx)
    out = jnp.reshape(xf + (q - xf), input_shape)
    return out, loss2[0, 0]


# speedup vs baseline: 1.1382x; 1.1382x over previous
import functools

import jax
import jax.numpy as jnp
from jax import lax
from jax.experimental import pallas as pl
from jax.experimental.pallas import tpu as pltpu
from jax.experimental.pallas import tpu_sc as plsc

D = 256
NE = 8192
NROW = 8192
RT = 1024
CT = 1024
NI = NROW // RT
NJ = NE // CT
BETA = 0.25


SUB = 64
NS = RT // SUB


def _dist_body(x_ref, e2x_ref, idx_ref, loss_ref, x2_ref, min_ref, arg_ref):
    i = pl.program_id(0)
    j = pl.program_id(1)

    @pl.when(j == 0)
    def _():
        xt = x_ref[...]
        x2 = jnp.sum(xt * xt, axis=1, keepdims=True)
        x2_ref[...] = jnp.transpose(x2)
        min_ref[...] = jnp.full((1, RT), jnp.inf, jnp.float32)

    e2x = e2x_ref[...]
    e2 = 0.25 * jnp.sum(e2x * e2x, axis=0, keepdims=True)
    e2t = jnp.transpose(e2)
    jcol = jnp.float32(j * CT)

    sim2 = lax.dot_general(e2x, x_ref[...], (((0,), (1,)), ((), ())),
                           preferred_element_type=jnp.float32)
    d = (x2_ref[...] + e2t) - sim2
    m = jnp.min(d, axis=0, keepdims=True)
    iota = lax.broadcasted_iota(jnp.int32, (CT, RT), 0).astype(jnp.float32)
    a = jnp.min(jnp.where(d == m, iota, jnp.float32(1e9)),
                axis=0, keepdims=True) + jcol
    prev_m = min_ref[...]
    upd = m < prev_m
    min_ref[...] = jnp.where(upd, m, prev_m)
    arg_ref[...] = jnp.where(upd, a, arg_ref[...])

    @pl.when(j == NJ - 1)
    def _():
        idx_ref[...] = arg_ref[...].astype(jnp.int32).reshape(1, 1, RT)
        tile_sum = jnp.sum(min_ref[...])
        prev = jnp.where(i == 0, jnp.zeros((1, 1), jnp.float32), loss_ref[...])
        tot = prev + tile_sum
        scale = jnp.float32((1.0 + BETA) / (NROW * D))
        loss_ref[...] = jnp.where(i == NI - 1, tot * scale, tot)


_dist_call = pl.pallas_call(
    _dist_body,
    grid=(NI, NJ),
    in_specs=[
        pl.BlockSpec((RT, D), lambda i, j: (i, 0)),
        pl.BlockSpec((D, CT), lambda i, j: (0, j)),
    ],
    out_specs=[
        pl.BlockSpec((1, 1, RT), lambda i, j: (i, 0, 0)),
        pl.BlockSpec((1, 1), lambda i, j: (0, 0)),
    ],
    out_shape=[
        jax.ShapeDtypeStruct((NI, 1, RT), jnp.int32),
        jax.ShapeDtypeStruct((1, 1), jnp.float32),
    ],
    scratch_shapes=[
        pltpu.VMEM((1, RT), jnp.float32),
        pltpu.VMEM((1, RT), jnp.float32),
        pltpu.VMEM((1, RT), jnp.float32),
    ],
    compiler_params=pltpu.CompilerParams(
        dimension_semantics=("arbitrary", "arbitrary"),
    ),
)


NW = 32
BPW = NROW // NW
CHUNK = 128
NCH = BPW // CHUNK

@functools.cache
def _sc_gather_call():
    mesh = plsc.VectorSubcoreMesh(core_axis_name="c", subcore_axis_name="s")

    @functools.partial(
        pl.kernel,
        mesh=mesh,
        out_type=jax.ShapeDtypeStruct((NROW, D), jnp.float32),
        scratch_types=[
            pltpu.VMEM((NCH, CHUNK), jnp.int32),
            pltpu.VMEM((BPW, D), jnp.float32),
            pltpu.SemaphoreType.DMA,
        ],
    )
    def _sc_gather(table_hbm, idx_hbm, out_hbm, idx_v, rows_v, sem):
        wid = lax.axis_index("s") * 2 + lax.axis_index("c")
        base = wid * BPW
        pltpu.sync_copy(idx_hbm.at[wid], idx_v)
        cps = [
            pltpu.async_copy(table_hbm.at[idx_v.at[k]],
                             rows_v.at[pl.ds(k * CHUNK, CHUNK)], sem)
            for k in range(NCH)
        ]
        for cp in cps:
            cp.wait()
        pltpu.sync_copy(rows_v, out_hbm.at[pl.ds(base, BPW)])

    return _sc_gather


def kernel(x, embeddings):
    input_shape = x.shape
    xf = jnp.reshape(x, (NROW, D))
    idx3, loss2 = _dist_call(xf, embeddings + embeddings)
    idx = jnp.reshape(idx3, (NW, NCH, CHUNK))
    table = embeddings.T
    q = _sc_gather_call()(table, idx)
    out = jnp.reshape(xf + (q - xf), input_shape)
    return out, loss2[0, 0]


# speedup vs baseline: 1.1771x; 1.0342x over previous
import functools

import jax
import jax.numpy as jnp
from jax import lax
from jax.experimental import pallas as pl
from jax.experimental.pallas import tpu as pltpu
from jax.experimental.pallas import tpu_sc as plsc

D = 256
NE = 8192
NROW = 8192
RT = 1024
CT = 2048
NI = NROW // RT
NJ = NE // CT
BETA = 0.25


SUB = 64
NS = RT // SUB


def _dist_body(x_ref, e2x_ref, idx_ref, loss_ref, x2_ref, min_ref, arg_ref):
    i = pl.program_id(0)
    j = pl.program_id(1)

    @pl.when(j == 0)
    def _():
        xt = x_ref[...]
        x2 = jnp.sum(xt * xt, axis=1, keepdims=True)
        x2_ref[...] = jnp.transpose(x2)
        min_ref[...] = jnp.full((1, RT), jnp.inf, jnp.float32)

    e2x = e2x_ref[...]
    e2 = 0.25 * jnp.sum(e2x * e2x, axis=0, keepdims=True)
    e2t = jnp.transpose(e2)
    jcol = jnp.float32(j * CT)

    sim2 = lax.dot_general(e2x, x_ref[...], (((0,), (1,)), ((), ())),
                           preferred_element_type=jnp.float32)
    d = (x2_ref[...] + e2t) - sim2
    m = jnp.min(d, axis=0, keepdims=True)
    iota = lax.broadcasted_iota(jnp.int32, (CT, RT), 0).astype(jnp.float32)
    a = jnp.min(jnp.where(d == m, iota, jnp.float32(1e9)),
                axis=0, keepdims=True) + jcol
    prev_m = min_ref[...]
    upd = m < prev_m
    min_ref[...] = jnp.where(upd, m, prev_m)
    arg_ref[...] = jnp.where(upd, a, arg_ref[...])

    @pl.when(j == NJ - 1)
    def _():
        idx_ref[...] = arg_ref[...].astype(jnp.int32).reshape(1, 1, RT)
        tile_sum = jnp.sum(min_ref[...])
        prev = jnp.where(i == 0, jnp.zeros((1, 1), jnp.float32), loss_ref[...])
        tot = prev + tile_sum
        scale = jnp.float32((1.0 + BETA) / (NROW * D))
        loss_ref[...] = jnp.where(i == NI - 1, tot * scale, tot)


_dist_call = pl.pallas_call(
    _dist_body,
    grid=(NI, NJ),
    in_specs=[
        pl.BlockSpec((RT, D), lambda i, j: (i, 0)),
        pl.BlockSpec((D, CT), lambda i, j: (0, j)),
    ],
    out_specs=[
        pl.BlockSpec((1, 1, RT), lambda i, j: (i, 0, 0)),
        pl.BlockSpec((1, 1), lambda i, j: (0, 0)),
    ],
    out_shape=[
        jax.ShapeDtypeStruct((NI, 1, RT), jnp.int32),
        jax.ShapeDtypeStruct((1, 1), jnp.float32),
    ],
    scratch_shapes=[
        pltpu.VMEM((1, RT), jnp.float32),
        pltpu.VMEM((1, RT), jnp.float32),
        pltpu.VMEM((1, RT), jnp.float32),
    ],
    compiler_params=pltpu.CompilerParams(
        dimension_semantics=("arbitrary", "arbitrary"),
    ),
)


NW = 32
BPW = NROW // NW
CHUNK = 128
NCH = BPW // CHUNK

@functools.cache
def _sc_gather_call():
    mesh = plsc.VectorSubcoreMesh(core_axis_name="c", subcore_axis_name="s")

    @functools.partial(
        pl.kernel,
        mesh=mesh,
        out_type=jax.ShapeDtypeStruct((NROW, D), jnp.float32),
        scratch_types=[
            pltpu.VMEM((NCH, CHUNK), jnp.int32),
            pltpu.VMEM((BPW, D), jnp.float32),
            pltpu.SemaphoreType.DMA,
        ],
    )
    def _sc_gather(table_hbm, idx_hbm, out_hbm, idx_v, rows_v, sem):
        wid = lax.axis_index("s") * 2 + lax.axis_index("c")
        base = wid * BPW
        pltpu.sync_copy(idx_hbm.at[wid], idx_v)
        cps = [
            pltpu.async_copy(table_hbm.at[idx_v.at[k]],
                             rows_v.at[pl.ds(k * CHUNK, CHUNK)], sem)
            for k in range(NCH)
        ]
        for cp in cps:
            cp.wait()
        pltpu.sync_copy(rows_v, out_hbm.at[pl.ds(base, BPW)])

    return _sc_gather


def kernel(x, embeddings):
    input_shape = x.shape
    xf = jnp.reshape(x, (NROW, D))
    idx3, loss2 = _dist_call(xf, embeddings + embeddings)
    idx = jnp.reshape(idx3, (NW, NCH, CHUNK))
    table = embeddings.T
    q = _sc_gather_call()(table, idx)
    out = jnp.reshape(xf + (q - xf), input_shape)
    return out, loss2[0, 0]


# speedup vs baseline: 1.2168x; 1.0337x over previous
import functools

import jax
import jax.numpy as jnp
from jax import lax
from jax.experimental import pallas as pl
from jax.experimental.pallas import tpu as pltpu
from jax.experimental.pallas import tpu_sc as plsc

D = 256
NE = 8192
NROW = 8192
RT = 1024
CT = 4096
NI = NROW // RT
NJ = NE // CT
BETA = 0.25


SUB = 64
NS = RT // SUB


def _dist_body(x_ref, e2x_ref, idx_ref, loss_ref, x2_ref, min_ref, arg_ref):
    i = pl.program_id(0)
    j = pl.program_id(1)

    @pl.when(j == 0)
    def _():
        xt = x_ref[...]
        x2 = jnp.sum(xt * xt, axis=1, keepdims=True)
        x2_ref[...] = jnp.transpose(x2)
        min_ref[...] = jnp.full((1, RT), jnp.inf, jnp.float32)

    e2x = e2x_ref[...]
    e2 = 0.25 * jnp.sum(e2x * e2x, axis=0, keepdims=True)
    e2t = jnp.transpose(e2)
    jcol = jnp.float32(j * CT)

    sim2 = lax.dot_general(e2x, x_ref[...], (((0,), (1,)), ((), ())),
                           preferred_element_type=jnp.float32)
    d = (x2_ref[...] + e2t) - sim2
    m = jnp.min(d, axis=0, keepdims=True)
    iota = lax.broadcasted_iota(jnp.int32, (CT, RT), 0).astype(jnp.float32)
    a = jnp.min(jnp.where(d == m, iota, jnp.float32(1e9)),
                axis=0, keepdims=True) + jcol
    prev_m = min_ref[...]
    upd = m < prev_m
    min_ref[...] = jnp.where(upd, m, prev_m)
    arg_ref[...] = jnp.where(upd, a, arg_ref[...])

    @pl.when(j == NJ - 1)
    def _():
        idx_ref[...] = arg_ref[...].astype(jnp.int32).reshape(1, 1, RT)
        tile_sum = jnp.sum(min_ref[...])
        prev = jnp.where(i == 0, jnp.zeros((1, 1), jnp.float32), loss_ref[...])
        tot = prev + tile_sum
        scale = jnp.float32((1.0 + BETA) / (NROW * D))
        loss_ref[...] = jnp.where(i == NI - 1, tot * scale, tot)


_dist_call = pl.pallas_call(
    _dist_body,
    grid=(NI, NJ),
    in_specs=[
        pl.BlockSpec((RT, D), lambda i, j: (i, 0)),
        pl.BlockSpec((D, CT), lambda i, j: (0, j)),
    ],
    out_specs=[
        pl.BlockSpec((1, 1, RT), lambda i, j: (i, 0, 0)),
        pl.BlockSpec((1, 1), lambda i, j: (0, 0)),
    ],
    out_shape=[
        jax.ShapeDtypeStruct((NI, 1, RT), jnp.int32),
        jax.ShapeDtypeStruct((1, 1), jnp.float32),
    ],
    scratch_shapes=[
        pltpu.VMEM((1, RT), jnp.float32),
        pltpu.VMEM((1, RT), jnp.float32),
        pltpu.VMEM((1, RT), jnp.float32),
    ],
    compiler_params=pltpu.CompilerParams(
        dimension_semantics=("arbitrary", "arbitrary"),
    ),
)


NW = 32
BPW = NROW // NW
CHUNK = 128
NCH = BPW // CHUNK

@functools.cache
def _sc_gather_call():
    mesh = plsc.VectorSubcoreMesh(core_axis_name="c", subcore_axis_name="s")

    @functools.partial(
        pl.kernel,
        mesh=mesh,
        out_type=jax.ShapeDtypeStruct((NROW, D), jnp.float32),
        scratch_types=[
            pltpu.VMEM((NCH, CHUNK), jnp.int32),
            pltpu.VMEM((BPW, D), jnp.float32),
            pltpu.SemaphoreType.DMA,
        ],
    )
    def _sc_gather(table_hbm, idx_hbm, out_hbm, idx_v, rows_v, sem):
        wid = lax.axis_index("s") * 2 + lax.axis_index("c")
        base = wid * BPW
        pltpu.sync_copy(idx_hbm.at[wid], idx_v)
        cps = [
            pltpu.async_copy(table_hbm.at[idx_v.at[k]],
                             rows_v.at[pl.ds(k * CHUNK, CHUNK)], sem)
            for k in range(NCH)
        ]
        for cp in cps:
            cp.wait()
        pltpu.sync_copy(rows_v, out_hbm.at[pl.ds(base, BPW)])

    return _sc_gather


def kernel(x, embeddings):
    input_shape = x.shape
    xf = jnp.reshape(x, (NROW, D))
    idx3, loss2 = _dist_call(xf, embeddings + embeddings)
    idx = jnp.reshape(idx3, (NW, NCH, CHUNK))
    table = embeddings.T
    q = _sc_gather_call()(table, idx)
    out = jnp.reshape(xf + (q - xf), input_shape)
    return out, loss2[0, 0]
